# baseline (speedup 1.0000x reference)
import jax
import jax.numpy as jnp
from jax import lax
from jax.experimental import pallas as pl
from jax.experimental.pallas import tpu as pltpu
from jax.experimental.pallas import tpu_sc as plsc

_U = 20000
_I = 20000
_G = 1000
_D = 64
_L = 2
_B = 4096
_M = 32
_TEMP = 0.2
_CL_W = 0.1

_NNZ = 96000
_NW = 32
_K = 128
_NCHUNK = 24
_NNZ_PAD = _NW * _NCHUNK * _K
_SEG = 2 * _G
_DP = 128

_ROWS_PER_BLK = 2000
_BATCH_TILE = 512


_ZW = 10
_ZROWS = _SEG // _ZW
_UCH = 9
_ICH = 18


def _seg_ring(emb_hbm, gidx_v, srow_v, vals_v, acc_sh, bufs, sems, ssems,
              nchunk):
    for b in range(2):
        pltpu.async_copy(emb_hbm.at[gidx_v.at[b]], bufs[b], sems[b])

    @pl.loop(0, nchunk, step=3)
    def _trip(c0):
        for b in range(3):
            c = c0 + b
            buf = bufs[b]
            p = (b + 2) % 3
            pltpu.make_async_copy(emb_hbm.at[gidx_v.at[c]], buf,
                                  sems[b]).wait()

            @pl.loop(0, _K // 16)
            def _scale(g):
                val16 = vals_v[c, pl.ds(g * 16, 16)]
                for l in range(16):
                    j = g * 16 + l
                    val = val16[l]
                    for c4 in range(_D // 16):
                        sl = pl.ds(c4 * 16, 16)
                        buf[j, sl] = buf[j, sl] * val

            pltpu.async_copy(buf, acc_sh.at[srow_v.at[c]], ssems[b],
                             add=True)

            @pl.when(c + 2 < nchunk)
            def _prefetch():
                @pl.when(c >= 1)
                def _drain_prev():
                    pltpu.make_async_copy(
                        bufs[p], acc_sh.at[srow_v.at[c]],
                        ssems[p]).wait()

                pltpu.async_copy(emb_hbm.at[gidx_v.at[c + 2]], bufs[p],
                                 sems[p])

    for b in range(3):
        pltpu.make_async_copy(bufs[b],
                              acc_sh.at[srow_v.at[nchunk - 3 + b]],
                              ssems[b]).wait()


def _segsum_body(embu_hbm, embi_hbm, gu_hbm, su_hbm, vu_hbm,
                 gi_hbm, si_hbm, vi_hbm, out_hbm,
                 gu_v, su_v, vu_v, gi_v, si_v, vi_v,
                 buf0, buf1, buf2, zbuf, acc_sh,
                 sem0, sem1, sem2, ssem0, ssem1, ssem2):
    cid = lax.axis_index("c")
    sid = lax.axis_index("s")
    wid = sid * 2 + cid
    bufs = (buf0, buf1, buf2)
    sems = (sem0, sem1, sem2)
    ssems = (ssem0, ssem1, ssem2)

    pltpu.sync_copy(gu_hbm.at[wid], gu_v)
    pltpu.sync_copy(su_hbm.at[wid], su_v)
    pltpu.sync_copy(vu_hbm.at[wid], vu_v)
    pltpu.sync_copy(gi_hbm.at[wid], gi_v)
    pltpu.sync_copy(si_hbm.at[wid], si_v)
    pltpu.sync_copy(vi_hbm.at[wid], vi_v)

    zeros16 = jnp.zeros((16,), jnp.float32)

    @pl.when(sid < _ZW)
    def _zero_acc():
        @pl.loop(0, _ZROWS)
        def _zero(r):
            for c8 in range(_DP // 16):
                zbuf[r, pl.ds(c8 * 16, 16)] = zeros16

        pltpu.sync_copy(zbuf, acc_sh.at[pl.ds(sid * _ZROWS, _ZROWS)])

    plsc.subcore_barrier()

    _seg_ring(embu_hbm, gu_v, su_v, vu_v, acc_sh, bufs, sems, ssems, _UCH)
    _seg_ring(embi_hbm, gi_v, si_v, vi_v, acc_sh, bufs, sems, ssems, _ICH)

    plsc.subcore_barrier()

    @pl.when(sid == 0)
    def _flush():
        pltpu.sync_copy(acc_sh, out_hbm.at[cid])


def _sc_segsum(emb_u, emb_i, uidx, iidx):
    return pl.kernel(
        _segsum_body,
        out_type=jax.ShapeDtypeStruct((2, _SEG, _DP), jnp.float32),
        mesh=plsc.VectorSubcoreMesh(core_axis_name="c", subcore_axis_name="s",
                                    num_cores=2, num_subcores=16),
        scratch_types=[
            pltpu.VMEM((_UCH, _K), jnp.int32),
            pltpu.VMEM((_UCH, _K), jnp.int32),
            pltpu.VMEM((_UCH, _K), jnp.float32),
            pltpu.VMEM((_ICH, _K), jnp.int32),
            pltpu.VMEM((_ICH, _K), jnp.int32),
            pltpu.VMEM((_ICH, _K), jnp.float32),
            pltpu.VMEM((_K, _DP), jnp.float32),
            pltpu.VMEM((_K, _DP), jnp.float32),
            pltpu.VMEM((_K, _DP), jnp.float32),
            pltpu.VMEM((_ZROWS, _DP), jnp.float32),
            pltpu.VMEM_SHARED((_SEG, _DP), jnp.float32),
            pltpu.SemaphoreType.DMA,
            pltpu.SemaphoreType.DMA,
            pltpu.SemaphoreType.DMA,
            pltpu.SemaphoreType.DMA,
            pltpu.SemaphoreType.DMA,
            pltpu.SemaphoreType.DMA,
        ],
    )(emb_u, emb_i, *uidx, *iidx)


def _prep_indices_half(cols, rows, vals, nchunk, nrows, row_offset):
    n = cols.shape[0]
    pad = _NW * nchunk * _K - n
    pad_idx = jnp.arange(pad, dtype=jnp.int32) % nrows
    pad_row = jnp.arange(pad, dtype=jnp.int32) % _SEG
    gidx = jnp.concatenate([cols.astype(jnp.int32), pad_idx]
                           ).reshape(_NW, nchunk, _K)
    srow = jnp.concatenate([rows.astype(jnp.int32) + row_offset, pad_row]
                           ).reshape(_NW, nchunk, _K)
    svals = jnp.concatenate([vals, jnp.zeros((pad,), jnp.float32)]
                            ).reshape(_NW, nchunk, _K)
    return gidx, srow, svals


_BW = _B // _NW
_MCH = 4
_NMCH = _BW // _MCH


def _memcen_body(fui_hbm, mem_hbm, cen_out,
                 mem_idx, buf0, buf1, cbuf, sem0, sem1):
    cid = lax.axis_index("c")
    sid = lax.axis_index("s")
    wid = sid * 2 + cid
    bufs = (buf0, buf1)
    sems = (sem0, sem1)

    pltpu.sync_copy(mem_hbm.at[wid], mem_idx)

    for b in range(2):
        pltpu.async_copy(fui_hbm.at[mem_idx.at[b]], bufs[b], sems[b])

    @pl.loop(0, _NMCH, step=2)
    def _mpair(c0):
        for b in range(2):
            c = c0 + b
            buf = bufs[b]
            pltpu.make_async_copy(fui_hbm.at[mem_idx.at[c]], buf,
                                  sems[b]).wait()
            for g4 in range(_MCH):
                base = g4 * _M
                for c4 in range(_D // 16):
                    sl = pl.ds(c4 * 16, 16)
                    amax = buf[base, sl]
                    amin = amax
                    for r in range(1, _M):
                        x = buf[base + r, sl]
                        amax = jnp.maximum(amax, x)
                        amin = jnp.minimum(amin, x)
                    cbuf[c * _MCH + g4, sl] = (amax + amin) * 0.5

            @pl.when(c < _NMCH - 2)
            def _prefetch():
                pltpu.async_copy(fui_hbm.at[mem_idx.at[c + 2]], buf, sems[b])

    pltpu.sync_copy(cbuf, cen_out.at[pl.ds(wid * _BW, _BW)])


def _sc_memcen(fui_u, members):
    return pl.kernel(
        _memcen_body,
        out_type=jax.ShapeDtypeStruct((_B, _DP), jnp.float32),
        cost_estimate=pl.CostEstimate(
            flops=2 * _B * _M * _D, transcendentals=0,
            bytes_accessed=_B * _M * _DP * 4 + _B * _DP * 4),
        mesh=plsc.VectorSubcoreMesh(core_axis_name="c", subcore_axis_name="s",
                                    num_cores=2, num_subcores=16),
        scratch_types=[
            pltpu.VMEM((_NMCH, _BW), jnp.int32),
            pltpu.VMEM((_BW, _DP), jnp.float32),
            pltpu.VMEM((_BW, _DP), jnp.float32),
            pltpu.VMEM((_BW, _DP), jnp.float32),
            pltpu.SemaphoreType.DMA,
            pltpu.SemaphoreType.DMA,
        ],
    )(fui_u, members)


def _pngb_body(fui_hbm, fg_hbm, pos_hbm, neg_hbm, grp_hbm,
               pos_out, neg_out, gb_out,
               png_idx, buf0, buf1, sem0, sem1):
    cid = lax.axis_index("c")
    sid = lax.axis_index("s")
    wid = sid * 2 + cid

    pltpu.sync_copy(pos_hbm.at[wid], png_idx.at[0])
    pltpu.sync_copy(neg_hbm.at[wid], png_idx.at[1])
    pltpu.sync_copy(grp_hbm.at[wid], png_idx.at[2])

    pltpu.async_copy(fui_hbm.at[png_idx.at[0]], buf0, sem0)
    pltpu.async_copy(fui_hbm.at[png_idx.at[1]], buf1, sem1)
    pltpu.make_async_copy(fui_hbm.at[png_idx.at[0]], buf0, sem0).wait()
    pltpu.sync_copy(buf0, pos_out.at[pl.ds(wid * _BW, _BW)])
    pltpu.async_copy(fg_hbm.at[png_idx.at[2]], buf0, sem0)
    pltpu.make_async_copy(fui_hbm.at[png_idx.at[1]], buf1, sem1).wait()
    pltpu.sync_copy(buf1, neg_out.at[pl.ds(wid * _BW, _BW)])
    pltpu.make_async_copy(fg_hbm.at[png_idx.at[2]], buf0, sem0).wait()
    pltpu.sync_copy(buf0, gb_out.at[pl.ds(wid * _BW, _BW)])


def _sc_pngb(fui_i, fgp, pos_idx, neg_idx, grp_idx):
    out = jax.ShapeDtypeStruct((_B, _DP), jnp.float32)
    return pl.kernel(
        _pngb_body,
        out_type=[out, out, out],
        mesh=plsc.VectorSubcoreMesh(core_axis_name="c", subcore_axis_name="s",
                                    num_cores=2, num_subcores=16),
        scratch_types=[
            pltpu.VMEM((3, _BW), jnp.int32),
            pltpu.VMEM((_BW, _DP), jnp.float32),
            pltpu.VMEM((_BW, _DP), jnp.float32),
            pltpu.SemaphoreType.DMA,
            pltpu.SemaphoreType.DMA,
        ],
    )(fui_i, fgp, pos_idx, neg_idx, grp_idx)


def _finalg_kernel(a_ref, x_ref, m1_ref, m2_ref, o_ref):
    ge = jnp.dot(a_ref[...], x_ref[...], preferred_element_type=jnp.float32)
    s = ge + m1_ref[...] + m2_ref[...]
    o_ref[...] = jnp.concatenate(
        [s, jnp.zeros((_G, _DP - _D), jnp.float32)], axis=1)


def _finalg_pad(overlap_graph, group_table, msg1, msg2):
    return pl.pallas_call(
        _finalg_kernel,
        out_shape=jax.ShapeDtypeStruct((_G, _DP), jnp.float32),
    )(overlap_graph, group_table, msg1, msg2)


def _msg_kernel(p_ref, w_ref, b_ref, o_ref):
    um = p_ref[0, :_G, :_D] + p_ref[1, :_G, :_D]
    im = p_ref[0, _G:, :_D] + p_ref[1, _G:, :_D]
    acc = jnp.dot(um, w_ref[:_D, :], preferred_element_type=jnp.float32)
    acc += jnp.dot(im, w_ref[_D:, :], preferred_element_type=jnp.float32)
    o_ref[...] = acc + b_ref[...]


def _msg_mm(parts, w, b):
    return pl.pallas_call(
        _msg_kernel,
        out_shape=jax.ShapeDtypeStruct((_G, _D), jnp.float32),
    )(parts, w, b.reshape(1, _D))


def _bigmm_pad_kernel(a_ref, x_ref, o_ref):
    mm = jnp.dot(a_ref[...], x_ref[...], preferred_element_type=jnp.float32)
    o_ref[...] = jnp.concatenate(
        [mm, jnp.zeros((mm.shape[0], _DP - _D), jnp.float32)], axis=1)


def _bigmm_pad_half(a, x, half):
    m_half = _U
    k = a.shape[1]
    off = half * (m_half // _ROWS_PER_BLK)
    return pl.pallas_call(
        _bigmm_pad_kernel,
        grid=(m_half // _ROWS_PER_BLK,),
        in_specs=[
            pl.BlockSpec((_ROWS_PER_BLK, k), lambda i: (i + off, 0)),
            pl.BlockSpec((k, _D), lambda i: (0, 0)),
        ],
        out_specs=pl.BlockSpec((_ROWS_PER_BLK, _DP), lambda i: (i, 0)),
        out_shape=jax.ShapeDtypeStruct((m_half, _DP), jnp.float32),
    )(a, x)


def _bigmm_add2_kernel(a_ref, x_ref, b1_ref, b2_ref, o_ref):
    s = (b1_ref[...] + b2_ref[..., :_D] +
         jnp.dot(a_ref[...], x_ref[...], preferred_element_type=jnp.float32))
    o_ref[...] = jnp.concatenate(
        [s, jnp.zeros((s.shape[0], _DP - _D), jnp.float32)], axis=1)


def _bigmm_add2_half(a, x, base1_half, base2_half, half):
    m_half = _U
    k = a.shape[1]
    off = half * (m_half // _ROWS_PER_BLK)
    return pl.pallas_call(
        _bigmm_add2_kernel,
        grid=(m_half // _ROWS_PER_BLK,),
        in_specs=[
            pl.BlockSpec((_ROWS_PER_BLK, k), lambda i: (i + off, 0)),
            pl.BlockSpec((k, _D), lambda i: (0, 0)),
            pl.BlockSpec((_ROWS_PER_BLK, _D), lambda i: (i, 0)),
            pl.BlockSpec((_ROWS_PER_BLK, _DP), lambda i: (i, 0)),
        ],
        out_specs=pl.BlockSpec((_ROWS_PER_BLK, _DP), lambda i: (i, 0)),
        out_shape=jax.ShapeDtypeStruct((m_half, _DP), jnp.float32),
    )(a, x, base1_half, base2_half)


def _batch_kernel(centers_ref, gb_ref, ipos_ref, ineg_ref,
                  pw1_ref, pb1_ref, pw2_ref, pb2_ref,
                  pred_ref, part_ref):
    i0 = pl.program_id(0)
    c = centers_ref[..., :_D]
    gb_all = gb_ref[..., :_D]
    gbt = gb_ref[pl.ds(i0 * _BATCH_TILE, _BATCH_TILE), :_D]

    v1 = c / (jnp.sqrt(jnp.sum(c * c, axis=1, keepdims=True)) + 1e-12)
    v2 = gb_all / (jnp.sqrt(jnp.sum(gb_all * gb_all, axis=1,
                                    keepdims=True)) + 1e-12)
    v2t = gbt / (jnp.sqrt(jnp.sum(gbt * gbt, axis=1, keepdims=True)) + 1e-12)

    scores = jnp.exp(jnp.dot(v1, v2.T, preferred_element_type=jnp.float32)
                     / _TEMP)
    ttl = jnp.sum(scores, axis=1)
    pos = jnp.exp(jnp.sum(v1 * v2t, axis=1) / _TEMP)
    cl_part = jnp.sum(jnp.log(ttl) - jnp.log(pos))

    def predict(x):
        h = jnp.dot(x, pw1_ref[...], preferred_element_type=jnp.float32)
        h = h + pb1_ref[...]
        h = jnp.where(h > 0, h, 0.01 * h)
        return jnp.dot(h, pw2_ref[...],
                       preferred_element_type=jnp.float32) + pb2_ref[...]

    spos = jax.nn.sigmoid(predict(gbt * ipos_ref[..., :_D]))
    sneg = jax.nn.sigmoid(predict(gbt * ineg_ref[..., :_D]))
    bpr_part = jnp.sum(jnp.log(1.0 + jnp.exp(sneg - spos)))

    pred_ref[...] = spos
    lane = jax.lax.broadcasted_iota(jnp.int32, (1, 128), 1)
    vec = jnp.where(lane == 0, cl_part,
                    jnp.where(lane == 1, bpr_part, 0.0))
    part_ref[...] = vec.reshape(1, 1, 128)


def _batch_stage(centers, g_b, i_pos, i_neg, pW1, pb1, pW2, pb2):
    nblk = _B // _BATCH_TILE
    tile = pl.BlockSpec((_BATCH_TILE, _DP), lambda i: (i, 0))
    full = pl.BlockSpec((_B, _DP), lambda i: (0, 0))
    pred, parts = pl.pallas_call(
        _batch_kernel,
        grid=(nblk,),
        in_specs=[
            tile, full, tile, tile,
            pl.BlockSpec((_D, 8), lambda i: (0, 0)),
            pl.BlockSpec((1, 8), lambda i: (0, 0)),
            pl.BlockSpec((8, 1), lambda i: (0, 0)),
            pl.BlockSpec((1, 1), lambda i: (0, 0)),
        ],
        out_specs=[
            pl.BlockSpec((_BATCH_TILE, 1), lambda i: (i, 0)),
            pl.BlockSpec((1, 1, 128), lambda i: (i, 0, 0)),
        ],
        out_shape=[
            jax.ShapeDtypeStruct((_B, 1), jnp.float32),
            jax.ShapeDtypeStruct((nblk, 1, 128), jnp.float32),
        ],
    )(centers, g_b, i_pos, i_neg, pW1, pb1.reshape(1, 8),
      pW2, pb2.reshape(1, 1))
    return pred, parts


def kernel(user_table, item_table, group_table, overlap_graph, full_hyper,
           uh_vals, ih_vals, agg_W, agg_b, pW1, pb1, pW2, pb2,
           group_inputs, pos_item_inputs, neg_item_inputs, members,
           uh_rows, uh_cols, ih_rows, ih_cols):
    zpad = jnp.zeros((_U, _DP - _D), jnp.float32)
    cat_u = jnp.concatenate([user_table, zpad], axis=1)
    cat_i = jnp.concatenate([item_table, zpad], axis=1)
    uidx = _prep_indices_half(uh_cols, uh_rows, uh_vals, _UCH, _U, 0)
    iidx = _prep_indices_half(ih_cols, ih_rows, ih_vals, _ICH, _I, _G)

    emb_u, emb_i = cat_u, cat_i
    msgs = []
    norm1_u = norm1_i = None
    final_u = final_i = None
    for l in range(_L):
        parts = _sc_segsum(emb_u, emb_i, uidx, iidx)
        msg = _msg_mm(parts, agg_W[l], agg_b[l])
        msgs.append(msg)
        if l == 0:
            norm1_u = _bigmm_pad_half(full_hyper, msg, 0)
            norm1_i = _bigmm_pad_half(full_hyper, msg, 1)
            emb_u, emb_i = norm1_u, norm1_i
        else:
            final_u = _bigmm_add2_half(full_hyper, msg, user_table,
                                       norm1_u, half=0)
            final_i = _bigmm_add2_half(full_hyper, msg, item_table,
                                       norm1_i, half=1)

    final_gp = _finalg_pad(overlap_graph, group_table, msgs[0], msgs[1])

    mem_idx = members.astype(jnp.int32).reshape(_NW, _NMCH, _BW)
    pos_idx = pos_item_inputs.astype(jnp.int32).reshape(_NW, _BW)
    neg_idx = neg_item_inputs.astype(jnp.int32).reshape(_NW, _BW)
    grp_idx = group_inputs.astype(jnp.int32).reshape(_NW, _BW)
    centers = _sc_memcen(final_u, mem_idx)
    i_pos, i_neg, g_b = _sc_pngb(final_i, final_gp, pos_idx, neg_idx,
                                 grp_idx)

    pred, parts = _batch_stage(centers, g_b, i_pos, i_neg, pW1, pb1, pW2, pb2)
    cl_loss = jnp.sum(parts[:, 0, 0]) / _B
    bpr_loss = jnp.sum(parts[:, 0, 1]) / _B
    loss = bpr_loss + cl_loss * _CL_W
    return (loss, pred)


# speedup vs baseline: 1.0275x; 1.0275x over previous
import jax
import jax.numpy as jnp
from jax import lax
from jax.experimental import pallas as pl
from jax.experimental.pallas import tpu as pltpu
from jax.experimental.pallas import tpu_sc as plsc

_U = 20000
_I = 20000
_G = 1000
_D = 64
_L = 2
_B = 4096
_M = 32
_TEMP = 0.2
_CL_W = 0.1

_NNZ = 96000
_NW = 32
_K = 128
_NCHUNK = 24
_NNZ_PAD = _NW * _NCHUNK * _K
_SEG = 2 * _G
_DP = 128

_ROWS_PER_BLK = 2000
_BATCH_TILE = 512


_ZROWS = 200
_UCH = 9
_ICH = 18


def _seg_ring(emb_hbm, gidx_v, srow_v, vals_v, acc_sh, bufs, sems, ssems,
              nchunk):
    for b in range(2):
        pltpu.async_copy(emb_hbm.at[gidx_v.at[b]], bufs[b], sems[b])

    @pl.loop(0, nchunk, step=3)
    def _trip(c0):
        for b in range(3):
            c = c0 + b
            buf = bufs[b]
            p = (b + 2) % 3
            pltpu.make_async_copy(emb_hbm.at[gidx_v.at[c]], buf,
                                  sems[b]).wait()

            @pl.loop(0, _K // 16)
            def _scale(g):
                val16 = vals_v[c, pl.ds(g * 16, 16)]
                for l in range(16):
                    j = g * 16 + l
                    val = val16[l]
                    for c4 in range(_D // 16):
                        sl = pl.ds(c4 * 16, 16)
                        buf[j, sl] = buf[j, sl] * val

            pltpu.async_copy(buf, acc_sh.at[srow_v.at[c]], ssems[b],
                             add=True)

            @pl.when(c + 2 < nchunk)
            def _prefetch():
                @pl.when(c >= 1)
                def _drain_prev():
                    pltpu.make_async_copy(
                        bufs[p], acc_sh.at[srow_v.at[c]],
                        ssems[p]).wait()

                pltpu.async_copy(emb_hbm.at[gidx_v.at[c + 2]], bufs[p],
                                 sems[p])

    for b in range(3):
        pltpu.make_async_copy(bufs[b],
                              acc_sh.at[srow_v.at[nchunk - 3 + b]],
                              ssems[b]).wait()


def _make_segsum_body(nchunk, nseg):
    def _segsum_body(emb_hbm, gidx_hbm, srow_hbm, vals_hbm, out_hbm,
                     gidx_v, srow_v, vals_v, buf0, buf1, buf2, zbuf, acc_sh,
                     sem0, sem1, sem2, ssem0, ssem1, ssem2):
        cid = lax.axis_index("c")
        sid = lax.axis_index("s")
        wid = sid * 2 + cid
        bufs = (buf0, buf1, buf2)
        sems = (sem0, sem1, sem2)
        ssems = (ssem0, ssem1, ssem2)

        pltpu.sync_copy(gidx_hbm.at[wid], gidx_v)
        pltpu.sync_copy(srow_hbm.at[wid], srow_v)
        pltpu.sync_copy(vals_hbm.at[wid], vals_v)

        zeros16 = jnp.zeros((16,), jnp.float32)
        zw = nseg // _ZROWS

        @pl.when(sid < zw)
        def _zero_acc():
            @pl.loop(0, _ZROWS)
            def _zero(r):
                for c8 in range(_DP // 16):
                    zbuf[r, pl.ds(c8 * 16, 16)] = zeros16

            pltpu.sync_copy(zbuf, acc_sh.at[pl.ds(sid * _ZROWS, _ZROWS)])

        plsc.subcore_barrier()

        _seg_ring(emb_hbm, gidx_v, srow_v, vals_v, acc_sh, bufs, sems,
                  ssems, nchunk)

        plsc.subcore_barrier()

        @pl.when(sid == 0)
        def _flush():
            pltpu.sync_copy(acc_sh, out_hbm.at[cid])

    return _segsum_body


def _sc_segsum_half(emb, gidx, srow, vals, nchunk):
    return pl.kernel(
        _make_segsum_body(nchunk, _G),
        out_type=jax.ShapeDtypeStruct((2, _G, _DP), jnp.float32),
        mesh=plsc.VectorSubcoreMesh(core_axis_name="c", subcore_axis_name="s",
                                    num_cores=2, num_subcores=16),
        scratch_types=[
            pltpu.VMEM((nchunk, _K), jnp.int32),
            pltpu.VMEM((nchunk, _K), jnp.int32),
            pltpu.VMEM((nchunk, _K), jnp.float32),
            pltpu.VMEM((_K, _DP), jnp.float32),
            pltpu.VMEM((_K, _DP), jnp.float32),
            pltpu.VMEM((_K, _DP), jnp.float32),
            pltpu.VMEM((_ZROWS, _DP), jnp.float32),
            pltpu.VMEM_SHARED((_G, _DP), jnp.float32),
            pltpu.SemaphoreType.DMA,
            pltpu.SemaphoreType.DMA,
            pltpu.SemaphoreType.DMA,
            pltpu.SemaphoreType.DMA,
            pltpu.SemaphoreType.DMA,
            pltpu.SemaphoreType.DMA,
        ],
    )(emb, gidx, srow, vals)


def _prep_indices_half(cols, rows, vals, nchunk, nrows):
    n = cols.shape[0]
    pad = _NW * nchunk * _K - n
    pad_idx = jnp.arange(pad, dtype=jnp.int32) % nrows
    pad_row = jnp.arange(pad, dtype=jnp.int32) % _G
    gidx = jnp.concatenate([cols.astype(jnp.int32), pad_idx]
                           ).reshape(_NW, nchunk, _K)
    srow = jnp.concatenate([rows.astype(jnp.int32), pad_row]
                           ).reshape(_NW, nchunk, _K)
    svals = jnp.concatenate([vals, jnp.zeros((pad,), jnp.float32)]
                            ).reshape(_NW, nchunk, _K)
    return gidx, srow, svals


_BW = _B // _NW
_MCH = 4
_NMCH = _BW // _MCH


def _memcen_body(fui_hbm, mem_hbm, cen_out,
                 mem_idx, buf0, buf1, cbuf, sem0, sem1):
    cid = lax.axis_index("c")
    sid = lax.axis_index("s")
    wid = sid * 2 + cid
    bufs = (buf0, buf1)
    sems = (sem0, sem1)

    pltpu.sync_copy(mem_hbm.at[wid], mem_idx)

    for b in range(2):
        pltpu.async_copy(fui_hbm.at[mem_idx.at[b]], bufs[b], sems[b])

    @pl.loop(0, _NMCH, step=2)
    def _mpair(c0):
        for b in range(2):
            c = c0 + b
            buf = bufs[b]
            pltpu.make_async_copy(fui_hbm.at[mem_idx.at[c]], buf,
                                  sems[b]).wait()
            for g4 in range(_MCH):
                base = g4 * _M
                for c4 in range(_D // 16):
                    sl = pl.ds(c4 * 16, 16)
                    amax = buf[base, sl]
                    amin = amax
                    for r in range(1, _M):
                        x = buf[base + r, sl]
                        amax = jnp.maximum(amax, x)
                        amin = jnp.minimum(amin, x)
                    cbuf[c * _MCH + g4, sl] = (amax + amin) * 0.5

            @pl.when(c < _NMCH - 2)
            def _prefetch():
                pltpu.async_copy(fui_hbm.at[mem_idx.at[c + 2]], buf, sems[b])

    pltpu.sync_copy(cbuf, cen_out.at[pl.ds(wid * _BW, _BW)])


def _sc_memcen(fui_u, members):
    return pl.kernel(
        _memcen_body,
        out_type=jax.ShapeDtypeStruct((_B, _DP), jnp.float32),
        cost_estimate=pl.CostEstimate(
            flops=2 * _B * _M * _D, transcendentals=0,
            bytes_accessed=_B * _M * _DP * 4 + _B * _DP * 4),
        mesh=plsc.VectorSubcoreMesh(core_axis_name="c", subcore_axis_name="s",
                                    num_cores=2, num_subcores=16),
        scratch_types=[
            pltpu.VMEM((_NMCH, _BW), jnp.int32),
            pltpu.VMEM((_BW, _DP), jnp.float32),
            pltpu.VMEM((_BW, _DP), jnp.float32),
            pltpu.VMEM((_BW, _DP), jnp.float32),
            pltpu.SemaphoreType.DMA,
            pltpu.SemaphoreType.DMA,
        ],
    )(fui_u, members)


def _pngb_body(fui_hbm, fg_hbm, pos_hbm, neg_hbm, grp_hbm,
               pos_out, neg_out, gb_out,
               png_idx, buf0, buf1, sem0, sem1):
    cid = lax.axis_index("c")
    sid = lax.axis_index("s")
    wid = sid * 2 + cid

    pltpu.sync_copy(pos_hbm.at[wid], png_idx.at[0])
    pltpu.sync_copy(neg_hbm.at[wid], png_idx.at[1])
    pltpu.sync_copy(grp_hbm.at[wid], png_idx.at[2])

    pltpu.async_copy(fui_hbm.at[png_idx.at[0]], buf0, sem0)
    pltpu.async_copy(fui_hbm.at[png_idx.at[1]], buf1, sem1)
    pltpu.make_async_copy(fui_hbm.at[png_idx.at[0]], buf0, sem0).wait()
    pltpu.sync_copy(buf0, pos_out.at[pl.ds(wid * _BW, _BW)])
    pltpu.async_copy(fg_hbm.at[png_idx.at[2]], buf0, sem0)
    pltpu.make_async_copy(fui_hbm.at[png_idx.at[1]], buf1, sem1).wait()
    pltpu.sync_copy(buf1, neg_out.at[pl.ds(wid * _BW, _BW)])
    pltpu.make_async_copy(fg_hbm.at[png_idx.at[2]], buf0, sem0).wait()
    pltpu.sync_copy(buf0, gb_out.at[pl.ds(wid * _BW, _BW)])


def _sc_pngb(fui_i, fgp, pos_idx, neg_idx, grp_idx):
    out = jax.ShapeDtypeStruct((_B, _DP), jnp.float32)
    return pl.kernel(
        _pngb_body,
        out_type=[out, out, out],
        mesh=plsc.VectorSubcoreMesh(core_axis_name="c", subcore_axis_name="s",
                                    num_cores=2, num_subcores=16),
        scratch_types=[
            pltpu.VMEM((3, _BW), jnp.int32),
            pltpu.VMEM((_BW, _DP), jnp.float32),
            pltpu.VMEM((_BW, _DP), jnp.float32),
            pltpu.SemaphoreType.DMA,
            pltpu.SemaphoreType.DMA,
        ],
    )(fui_i, fgp, pos_idx, neg_idx, grp_idx)


def _finalg_kernel(a_ref, x_ref, m1_ref, m2_ref, o_ref):
    ge = jnp.dot(a_ref[...], x_ref[...], preferred_element_type=jnp.float32)
    s = ge + m1_ref[...] + m2_ref[...]
    o_ref[...] = jnp.concatenate(
        [s, jnp.zeros((_G, _DP - _D), jnp.float32)], axis=1)


def _finalg_pad(overlap_graph, group_table, msg1, msg2):
    return pl.pallas_call(
        _finalg_kernel,
        out_shape=jax.ShapeDtypeStruct((_G, _DP), jnp.float32),
    )(overlap_graph, group_table, msg1, msg2)


def _msg_kernel(pu_ref, pi_ref, w_ref, b_ref, o_ref):
    um = pu_ref[0, :, :_D] + pu_ref[1, :, :_D]
    im = pi_ref[0, :, :_D] + pi_ref[1, :, :_D]
    acc = jnp.dot(um, w_ref[:_D, :], preferred_element_type=jnp.float32)
    acc += jnp.dot(im, w_ref[_D:, :], preferred_element_type=jnp.float32)
    o_ref[...] = acc + b_ref[...]


def _msg_mm(parts_u, parts_i, w, b):
    return pl.pallas_call(
        _msg_kernel,
        out_shape=jax.ShapeDtypeStruct((_G, _D), jnp.float32),
    )(parts_u, parts_i, w, b.reshape(1, _D))


def _bigmm_pad_kernel(a_ref, x_ref, o_ref):
    mm = jnp.dot(a_ref[...], x_ref[...], preferred_element_type=jnp.float32)
    o_ref[...] = jnp.concatenate(
        [mm, jnp.zeros((mm.shape[0], _DP - _D), jnp.float32)], axis=1)


def _bigmm_pad_half(a, x, half):
    m_half = _U
    k = a.shape[1]
    off = half * (m_half // _ROWS_PER_BLK)
    return pl.pallas_call(
        _bigmm_pad_kernel,
        grid=(m_half // _ROWS_PER_BLK,),
        in_specs=[
            pl.BlockSpec((_ROWS_PER_BLK, k), lambda i: (i + off, 0)),
            pl.BlockSpec((k, _D), lambda i: (0, 0)),
        ],
        out_specs=pl.BlockSpec((_ROWS_PER_BLK, _DP), lambda i: (i, 0)),
        out_shape=jax.ShapeDtypeStruct((m_half, _DP), jnp.float32),
    )(a, x)


def _bigmm_add2_kernel(a_ref, x_ref, b1_ref, b2_ref, o_ref):
    s = (b1_ref[...] + b2_ref[..., :_D] +
         jnp.dot(a_ref[...], x_ref[...], preferred_element_type=jnp.float32))
    o_ref[...] = jnp.concatenate(
        [s, jnp.zeros((s.shape[0], _DP - _D), jnp.float32)], axis=1)


def _bigmm_add2_half(a, x, base1_half, base2_half, half):
    m_half = _U
    k = a.shape[1]
    off = half * (m_half // _ROWS_PER_BLK)
    return pl.pallas_call(
        _bigmm_add2_kernel,
        grid=(m_half // _ROWS_PER_BLK,),
        in_specs=[
            pl.BlockSpec((_ROWS_PER_BLK, k), lambda i: (i + off, 0)),
            pl.BlockSpec((k, _D), lambda i: (0, 0)),
            pl.BlockSpec((_ROWS_PER_BLK, _D), lambda i: (i, 0)),
            pl.BlockSpec((_ROWS_PER_BLK, _DP), lambda i: (i, 0)),
        ],
        out_specs=pl.BlockSpec((_ROWS_PER_BLK, _DP), lambda i: (i, 0)),
        out_shape=jax.ShapeDtypeStruct((m_half, _DP), jnp.float32),
    )(a, x, base1_half, base2_half)


def _batch_kernel(centers_ref, gb_ref, ipos_ref, ineg_ref,
                  pw1_ref, pb1_ref, pw2_ref, pb2_ref,
                  pred_ref, part_ref):
    i0 = pl.program_id(0)
    c = centers_ref[..., :_D]
    gb_all = gb_ref[..., :_D]
    gbt = gb_ref[pl.ds(i0 * _BATCH_TILE, _BATCH_TILE), :_D]

    v1 = c / (jnp.sqrt(jnp.sum(c * c, axis=1, keepdims=True)) + 1e-12)
    v2 = gb_all / (jnp.sqrt(jnp.sum(gb_all * gb_all, axis=1,
                                    keepdims=True)) + 1e-12)
    v2t = gbt / (jnp.sqrt(jnp.sum(gbt * gbt, axis=1, keepdims=True)) + 1e-12)

    scores = jnp.exp(jnp.dot(v1, v2.T, preferred_element_type=jnp.float32)
                     / _TEMP)
    ttl = jnp.sum(scores, axis=1)
    pos = jnp.exp(jnp.sum(v1 * v2t, axis=1) / _TEMP)
    cl_part = jnp.sum(jnp.log(ttl) - jnp.log(pos))

    def predict(x):
        h = jnp.dot(x, pw1_ref[...], preferred_element_type=jnp.float32)
        h = h + pb1_ref[...]
        h = jnp.where(h > 0, h, 0.01 * h)
        return jnp.dot(h, pw2_ref[...],
                       preferred_element_type=jnp.float32) + pb2_ref[...]

    spos = jax.nn.sigmoid(predict(gbt * ipos_ref[..., :_D]))
    sneg = jax.nn.sigmoid(predict(gbt * ineg_ref[..., :_D]))
    bpr_part = jnp.sum(jnp.log(1.0 + jnp.exp(sneg - spos)))

    pred_ref[...] = spos
    lane = jax.lax.broadcasted_iota(jnp.int32, (1, 128), 1)
    vec = jnp.where(lane == 0, cl_part,
                    jnp.where(lane == 1, bpr_part, 0.0))
    part_ref[...] = vec.reshape(1, 1, 128)


def _batch_stage(centers, g_b, i_pos, i_neg, pW1, pb1, pW2, pb2):
    nblk = _B // _BATCH_TILE
    tile = pl.BlockSpec((_BATCH_TILE, _DP), lambda i: (i, 0))
    full = pl.BlockSpec((_B, _DP), lambda i: (0, 0))
    pred, parts = pl.pallas_call(
        _batch_kernel,
        grid=(nblk,),
        in_specs=[
            tile, full, tile, tile,
            pl.BlockSpec((_D, 8), lambda i: (0, 0)),
            pl.BlockSpec((1, 8), lambda i: (0, 0)),
            pl.BlockSpec((8, 1), lambda i: (0, 0)),
            pl.BlockSpec((1, 1), lambda i: (0, 0)),
        ],
        out_specs=[
            pl.BlockSpec((_BATCH_TILE, 1), lambda i: (i, 0)),
            pl.BlockSpec((1, 1, 128), lambda i: (i, 0, 0)),
        ],
        out_shape=[
            jax.ShapeDtypeStruct((_B, 1), jnp.float32),
            jax.ShapeDtypeStruct((nblk, 1, 128), jnp.float32),
        ],
    )(centers, g_b, i_pos, i_neg, pW1, pb1.reshape(1, 8),
      pW2, pb2.reshape(1, 1))
    return pred, parts


def kernel(user_table, item_table, group_table, overlap_graph, full_hyper,
           uh_vals, ih_vals, agg_W, agg_b, pW1, pb1, pW2, pb2,
           group_inputs, pos_item_inputs, neg_item_inputs, members,
           uh_rows, uh_cols, ih_rows, ih_cols):
    zpad = jnp.zeros((_U, _DP - _D), jnp.float32)
    cat_u = jnp.concatenate([user_table, zpad], axis=1)
    cat_i = jnp.concatenate([item_table, zpad], axis=1)
    gu_idx, su_row, su_val = _prep_indices_half(uh_cols, uh_rows, uh_vals,
                                                _UCH, _U)
    gi_idx, si_row, si_val = _prep_indices_half(ih_cols, ih_rows, ih_vals,
                                                _ICH, _I)

    emb_u, emb_i = cat_u, cat_i
    msgs = []
    norm1_u = norm1_i = None
    final_u = final_i = None
    for l in range(_L):
        parts_u = _sc_segsum_half(emb_u, gu_idx, su_row, su_val, _UCH)
        parts_i = _sc_segsum_half(emb_i, gi_idx, si_row, si_val, _ICH)
        msg = _msg_mm(parts_u, parts_i, agg_W[l], agg_b[l])
        msgs.append(msg)
        if l == 0:
            norm1_u = _bigmm_pad_half(full_hyper, msg, 0)
            norm1_i = _bigmm_pad_half(full_hyper, msg, 1)
            emb_u, emb_i = norm1_u, norm1_i
        else:
            final_u = _bigmm_add2_half(full_hyper, msg, user_table,
                                       norm1_u, half=0)
            final_i = _bigmm_add2_half(full_hyper, msg, item_table,
                                       norm1_i, half=1)

    final_gp = _finalg_pad(overlap_graph, group_table, msgs[0], msgs[1])

    mem_idx = members.astype(jnp.int32).reshape(_NW, _NMCH, _BW)
    pos_idx = pos_item_inputs.astype(jnp.int32).reshape(_NW, _BW)
    neg_idx = neg_item_inputs.astype(jnp.int32).reshape(_NW, _BW)
    grp_idx = group_inputs.astype(jnp.int32).reshape(_NW, _BW)
    centers = _sc_memcen(final_u, mem_idx)
    i_pos, i_neg, g_b = _sc_pngb(final_i, final_gp, pos_idx, neg_idx,
                                 grp_idx)

    pred, parts = _batch_stage(centers, g_b, i_pos, i_neg, pW1, pb1, pW2, pb2)
    cl_loss = jnp.sum(parts[:, 0, 0]) / _B
    bpr_loss = jnp.sum(parts[:, 0, 1]) / _B
    loss = bpr_loss + cl_loss * _CL_W
    return (loss, pred)


# speedup vs baseline: 1.0379x; 1.0101x over previous
import jax
import jax.numpy as jnp
from jax import lax
from jax.experimental import pallas as pl
from jax.experimental.pallas import tpu as pltpu
from jax.experimental.pallas import tpu_sc as plsc

_U = 20000
_I = 20000
_G = 1000
_D = 64
_L = 2
_B = 4096
_M = 32
_TEMP = 0.2
_CL_W = 0.1

_NNZ = 96000
_NW = 32
_K = 128
_NCHUNK = 24
_NNZ_PAD = _NW * _NCHUNK * _K
_SEG = 2 * _G
_DP = 128

_ROWS_PER_BLK = 2000
_BATCH_TILE = 1024


_ZROWS = 200
_UCH = 9
_ICH = 18


def _seg_ring(emb_hbm, gidx_v, srow_v, vals_v, acc_sh, bufs, sems, ssems,
              nchunk):
    for b in range(2):
        pltpu.async_copy(emb_hbm.at[gidx_v.at[b]], bufs[b], sems[b])

    @pl.loop(0, nchunk, step=3)
    def _trip(c0):
        for b in range(3):
            c = c0 + b
            buf = bufs[b]
            p = (b + 2) % 3
            pltpu.make_async_copy(emb_hbm.at[gidx_v.at[c]], buf,
                                  sems[b]).wait()

            @pl.loop(0, _K // 16)
            def _scale(g):
                val16 = vals_v[c, pl.ds(g * 16, 16)]
                for l in range(16):
                    j = g * 16 + l
                    val = val16[l]
                    for c4 in range(_D // 16):
                        sl = pl.ds(c4 * 16, 16)
                        buf[j, sl] = buf[j, sl] * val

            pltpu.async_copy(buf, acc_sh.at[srow_v.at[c]], ssems[b],
                             add=True)

            @pl.when(c + 2 < nchunk)
            def _prefetch():
                @pl.when(c >= 1)
                def _drain_prev():
                    pltpu.make_async_copy(
                        bufs[p], acc_sh.at[srow_v.at[c]],
                        ssems[p]).wait()

                pltpu.async_copy(emb_hbm.at[gidx_v.at[c + 2]], bufs[p],
                                 sems[p])

    for b in range(3):
        pltpu.make_async_copy(bufs[b],
                              acc_sh.at[srow_v.at[nchunk - 3 + b]],
                              ssems[b]).wait()


def _make_segsum_body(nchunk, nseg):
    def _segsum_body(emb_hbm, gidx_hbm, srow_hbm, vals_hbm, out_hbm,
                     gidx_v, srow_v, vals_v, buf0, buf1, buf2, zbuf, acc_sh,
                     sem0, sem1, sem2, ssem0, ssem1, ssem2):
        cid = lax.axis_index("c")
        sid = lax.axis_index("s")
        wid = sid * 2 + cid
        bufs = (buf0, buf1, buf2)
        sems = (sem0, sem1, sem2)
        ssems = (ssem0, ssem1, ssem2)

        pltpu.sync_copy(gidx_hbm.at[wid], gidx_v)
        pltpu.sync_copy(srow_hbm.at[wid], srow_v)
        pltpu.sync_copy(vals_hbm.at[wid], vals_v)

        zeros16 = jnp.zeros((16,), jnp.float32)
        zw = nseg // _ZROWS

        @pl.when(sid < zw)
        def _zero_acc():
            @pl.loop(0, _ZROWS)
            def _zero(r):
                for c8 in range(_DP // 16):
                    zbuf[r, pl.ds(c8 * 16, 16)] = zeros16

            pltpu.sync_copy(zbuf, acc_sh.at[pl.ds(sid * _ZROWS, _ZROWS)])

        plsc.subcore_barrier()

        _seg_ring(emb_hbm, gidx_v, srow_v, vals_v, acc_sh, bufs, sems,
                  ssems, nchunk)

        plsc.subcore_barrier()

        @pl.when(sid == 0)
        def _flush():
            pltpu.sync_copy(acc_sh, out_hbm.at[cid])

    return _segsum_body


def _sc_segsum_half(emb, gidx, srow, vals, nchunk):
    return pl.kernel(
        _make_segsum_body(nchunk, _G),
        out_type=jax.ShapeDtypeStruct((2, _G, _DP), jnp.float32),
        mesh=plsc.VectorSubcoreMesh(core_axis_name="c", subcore_axis_name="s",
                                    num_cores=2, num_subcores=16),
        scratch_types=[
            pltpu.VMEM((nchunk, _K), jnp.int32),
            pltpu.VMEM((nchunk, _K), jnp.int32),
            pltpu.VMEM((nchunk, _K), jnp.float32),
            pltpu.VMEM((_K, _DP), jnp.float32),
            pltpu.VMEM((_K, _DP), jnp.float32),
            pltpu.VMEM((_K, _DP), jnp.float32),
            pltpu.VMEM((_ZROWS, _DP), jnp.float32),
            pltpu.VMEM_SHARED((_G, _DP), jnp.float32),
            pltpu.SemaphoreType.DMA,
            pltpu.SemaphoreType.DMA,
            pltpu.SemaphoreType.DMA,
            pltpu.SemaphoreType.DMA,
            pltpu.SemaphoreType.DMA,
            pltpu.SemaphoreType.DMA,
        ],
    )(emb, gidx, srow, vals)


def _prep_indices_half(cols, rows, vals, nchunk, nrows):
    n = cols.shape[0]
    pad = _NW * nchunk * _K - n
    pad_idx = jnp.arange(pad, dtype=jnp.int32) % nrows
    pad_row = jnp.arange(pad, dtype=jnp.int32) % _G
    gidx = jnp.concatenate([cols.astype(jnp.int32), pad_idx]
                           ).reshape(_NW, nchunk, _K)
    srow = jnp.concatenate([rows.astype(jnp.int32), pad_row]
                           ).reshape(_NW, nchunk, _K)
    svals = jnp.concatenate([vals, jnp.zeros((pad,), jnp.float32)]
                            ).reshape(_NW, nchunk, _K)
    return gidx, srow, svals


_BW = _B // _NW
_MCH = 4
_NMCH = _BW // _MCH


def _memcen_body(fui_hbm, mem_hbm, cen_out,
                 mem_idx, buf0, buf1, cbuf, sem0, sem1):
    cid = lax.axis_index("c")
    sid = lax.axis_index("s")
    wid = sid * 2 + cid
    bufs = (buf0, buf1)
    sems = (sem0, sem1)

    pltpu.sync_copy(mem_hbm.at[wid], mem_idx)

    for b in range(2):
        pltpu.async_copy(fui_hbm.at[mem_idx.at[b]], bufs[b], sems[b])

    @pl.loop(0, _NMCH, step=2)
    def _mpair(c0):
        for b in range(2):
            c = c0 + b
            buf = bufs[b]
            pltpu.make_async_copy(fui_hbm.at[mem_idx.at[c]], buf,
                                  sems[b]).wait()
            for g4 in range(_MCH):
                base = g4 * _M
                for c4 in range(_D // 16):
                    sl = pl.ds(c4 * 16, 16)
                    amax = buf[base, sl]
                    amin = amax
                    for r in range(1, _M):
                        x = buf[base + r, sl]
                        amax = jnp.maximum(amax, x)
                        amin = jnp.minimum(amin, x)
                    cbuf[c * _MCH + g4, sl] = (amax + amin) * 0.5

            @pl.when(c < _NMCH - 2)
            def _prefetch():
                pltpu.async_copy(fui_hbm.at[mem_idx.at[c + 2]], buf, sems[b])

    pltpu.sync_copy(cbuf, cen_out.at[pl.ds(wid * _BW, _BW)])


def _sc_memcen(fui_u, members):
    return pl.kernel(
        _memcen_body,
        out_type=jax.ShapeDtypeStruct((_B, _DP), jnp.float32),
        cost_estimate=pl.CostEstimate(
            flops=2 * _B * _M * _D, transcendentals=0,
            bytes_accessed=_B * _M * _DP * 4 + _B * _DP * 4),
        mesh=plsc.VectorSubcoreMesh(core_axis_name="c", subcore_axis_name="s",
                                    num_cores=2, num_subcores=16),
        scratch_types=[
            pltpu.VMEM((_NMCH, _BW), jnp.int32),
            pltpu.VMEM((_BW, _DP), jnp.float32),
            pltpu.VMEM((_BW, _DP), jnp.float32),
            pltpu.VMEM((_BW, _DP), jnp.float32),
            pltpu.SemaphoreType.DMA,
            pltpu.SemaphoreType.DMA,
        ],
    )(fui_u, members)


def _pngb_body(fui_hbm, fg_hbm, pos_hbm, neg_hbm, grp_hbm,
               pos_out, neg_out, gb_out,
               png_idx, buf0, buf1, sem0, sem1):
    cid = lax.axis_index("c")
    sid = lax.axis_index("s")
    wid = sid * 2 + cid

    pltpu.sync_copy(pos_hbm.at[wid], png_idx.at[0])
    pltpu.sync_copy(neg_hbm.at[wid], png_idx.at[1])
    pltpu.sync_copy(grp_hbm.at[wid], png_idx.at[2])

    pltpu.async_copy(fui_hbm.at[png_idx.at[0]], buf0, sem0)
    pltpu.async_copy(fui_hbm.at[png_idx.at[1]], buf1, sem1)
    pltpu.make_async_copy(fui_hbm.at[png_idx.at[0]], buf0, sem0).wait()
    pltpu.sync_copy(buf0, pos_out.at[pl.ds(wid * _BW, _BW)])
    pltpu.async_copy(fg_hbm.at[png_idx.at[2]], buf0, sem0)
    pltpu.make_async_copy(fui_hbm.at[png_idx.at[1]], buf1, sem1).wait()
    pltpu.sync_copy(buf1, neg_out.at[pl.ds(wid * _BW, _BW)])
    pltpu.make_async_copy(fg_hbm.at[png_idx.at[2]], buf0, sem0).wait()
    pltpu.sync_copy(buf0, gb_out.at[pl.ds(wid * _BW, _BW)])


def _sc_pngb(fui_i, fgp, pos_idx, neg_idx, grp_idx):
    out = jax.ShapeDtypeStruct((_B, _DP), jnp.float32)
    return pl.kernel(
        _pngb_body,
        out_type=[out, out, out],
        mesh=plsc.VectorSubcoreMesh(core_axis_name="c", subcore_axis_name="s",
                                    num_cores=2, num_subcores=16),
        scratch_types=[
            pltpu.VMEM((3, _BW), jnp.int32),
            pltpu.VMEM((_BW, _DP), jnp.float32),
            pltpu.VMEM((_BW, _DP), jnp.float32),
            pltpu.SemaphoreType.DMA,
            pltpu.SemaphoreType.DMA,
        ],
    )(fui_i, fgp, pos_idx, neg_idx, grp_idx)


def _finalg_kernel(a_ref, x_ref, m1_ref, m2_ref, o_ref):
    ge = jnp.dot(a_ref[...], x_ref[...], preferred_element_type=jnp.float32)
    s = ge + m1_ref[...] + m2_ref[...]
    o_ref[...] = jnp.concatenate(
        [s, jnp.zeros((_G, _DP - _D), jnp.float32)], axis=1)


def _finalg_pad(overlap_graph, group_table, msg1, msg2):
    return pl.pallas_call(
        _finalg_kernel,
        out_shape=jax.ShapeDtypeStruct((_G, _DP), jnp.float32),
    )(overlap_graph, group_table, msg1, msg2)


def _msg_kernel(pu_ref, pi_ref, w_ref, b_ref, o_ref):
    um = pu_ref[0, :, :_D] + pu_ref[1, :, :_D]
    im = pi_ref[0, :, :_D] + pi_ref[1, :, :_D]
    acc = jnp.dot(um, w_ref[:_D, :], preferred_element_type=jnp.float32)
    acc += jnp.dot(im, w_ref[_D:, :], preferred_element_type=jnp.float32)
    o_ref[...] = acc + b_ref[...]


def _msg_mm(parts_u, parts_i, w, b):
    return pl.pallas_call(
        _msg_kernel,
        out_shape=jax.ShapeDtypeStruct((_G, _D), jnp.float32),
    )(parts_u, parts_i, w, b.reshape(1, _D))


def _bigmm_pad_kernel(a_ref, x_ref, o_ref):
    mm = jnp.dot(a_ref[...], x_ref[...], preferred_element_type=jnp.float32)
    o_ref[...] = jnp.concatenate(
        [mm, jnp.zeros((mm.shape[0], _DP - _D), jnp.float32)], axis=1)


def _bigmm_pad_half(a, x, half):
    m_half = _U
    k = a.shape[1]
    off = half * (m_half // _ROWS_PER_BLK)
    return pl.pallas_call(
        _bigmm_pad_kernel,
        grid=(m_half // _ROWS_PER_BLK,),
        in_specs=[
            pl.BlockSpec((_ROWS_PER_BLK, k), lambda i: (i + off, 0)),
            pl.BlockSpec((k, _D), lambda i: (0, 0)),
        ],
        out_specs=pl.BlockSpec((_ROWS_PER_BLK, _DP), lambda i: (i, 0)),
        out_shape=jax.ShapeDtypeStruct((m_half, _DP), jnp.float32),
    )(a, x)


def _bigmm_add2_kernel(a_ref, x_ref, b1_ref, b2_ref, o_ref):
    s = (b1_ref[...] + b2_ref[..., :_D] +
         jnp.dot(a_ref[...], x_ref[...], preferred_element_type=jnp.float32))
    o_ref[...] = jnp.concatenate(
        [s, jnp.zeros((s.shape[0], _DP - _D), jnp.float32)], axis=1)


def _bigmm_add2_half(a, x, base1_half, base2_half, half):
    m_half = _U
    k = a.shape[1]
    off = half * (m_half // _ROWS_PER_BLK)
    return pl.pallas_call(
        _bigmm_add2_kernel,
        grid=(m_half // _ROWS_PER_BLK,),
        in_specs=[
            pl.BlockSpec((_ROWS_PER_BLK, k), lambda i: (i + off, 0)),
            pl.BlockSpec((k, _D), lambda i: (0, 0)),
            pl.BlockSpec((_ROWS_PER_BLK, _D), lambda i: (i, 0)),
            pl.BlockSpec((_ROWS_PER_BLK, _DP), lambda i: (i, 0)),
        ],
        out_specs=pl.BlockSpec((_ROWS_PER_BLK, _DP), lambda i: (i, 0)),
        out_shape=jax.ShapeDtypeStruct((m_half, _DP), jnp.float32),
    )(a, x, base1_half, base2_half)


def _batch_kernel(centers_ref, gb_ref, ipos_ref, ineg_ref,
                  pw1_ref, pb1_ref, pw2_ref, pb2_ref,
                  pred_ref, part_ref):
    i0 = pl.program_id(0)
    c = centers_ref[..., :_D]
    gb_all = gb_ref[..., :_D]
    gbt = gb_ref[pl.ds(i0 * _BATCH_TILE, _BATCH_TILE), :_D]

    v1 = c / (jnp.sqrt(jnp.sum(c * c, axis=1, keepdims=True)) + 1e-12)
    v2 = gb_all / (jnp.sqrt(jnp.sum(gb_all * gb_all, axis=1,
                                    keepdims=True)) + 1e-12)
    v2t = gbt / (jnp.sqrt(jnp.sum(gbt * gbt, axis=1, keepdims=True)) + 1e-12)

    scores = jnp.exp(jnp.dot(v1, v2.T, preferred_element_type=jnp.float32)
                     / _TEMP)
    ttl = jnp.sum(scores, axis=1)
    pos = jnp.exp(jnp.sum(v1 * v2t, axis=1) / _TEMP)
    cl_part = jnp.sum(jnp.log(ttl) - jnp.log(pos))

    def predict(x):
        h = jnp.dot(x, pw1_ref[...], preferred_element_type=jnp.float32)
        h = h + pb1_ref[...]
        h = jnp.where(h > 0, h, 0.01 * h)
        return jnp.dot(h, pw2_ref[...],
                       preferred_element_type=jnp.float32) + pb2_ref[...]

    spos = jax.nn.sigmoid(predict(gbt * ipos_ref[..., :_D]))
    sneg = jax.nn.sigmoid(predict(gbt * ineg_ref[..., :_D]))
    bpr_part = jnp.sum(jnp.log(1.0 + jnp.exp(sneg - spos)))

    pred_ref[...] = spos
    lane = jax.lax.broadcasted_iota(jnp.int32, (1, 128), 1)
    vec = jnp.where(lane == 0, cl_part,
                    jnp.where(lane == 1, bpr_part, 0.0))
    part_ref[...] = vec.reshape(1, 1, 128)


def _batch_stage(centers, g_b, i_pos, i_neg, pW1, pb1, pW2, pb2):
    nblk = _B // _BATCH_TILE
    tile = pl.BlockSpec((_BATCH_TILE, _DP), lambda i: (i, 0))
    full = pl.BlockSpec((_B, _DP), lambda i: (0, 0))
    pred, parts = pl.pallas_call(
        _batch_kernel,
        grid=(nblk,),
        in_specs=[
            tile, full, tile, tile,
            pl.BlockSpec((_D, 8), lambda i: (0, 0)),
            pl.BlockSpec((1, 8), lambda i: (0, 0)),
            pl.BlockSpec((8, 1), lambda i: (0, 0)),
            pl.BlockSpec((1, 1), lambda i: (0, 0)),
        ],
        out_specs=[
            pl.BlockSpec((_BATCH_TILE, 1), lambda i: (i, 0)),
            pl.BlockSpec((1, 1, 128), lambda i: (i, 0, 0)),
        ],
        out_shape=[
            jax.ShapeDtypeStruct((_B, 1), jnp.float32),
            jax.ShapeDtypeStruct((nblk, 1, 128), jnp.float32),
        ],
    )(centers, g_b, i_pos, i_neg, pW1, pb1.reshape(1, 8),
      pW2, pb2.reshape(1, 1))
    return pred, parts


def kernel(user_table, item_table, group_table, overlap_graph, full_hyper,
           uh_vals, ih_vals, agg_W, agg_b, pW1, pb1, pW2, pb2,
           group_inputs, pos_item_inputs, neg_item_inputs, members,
           uh_rows, uh_cols, ih_rows, ih_cols):
    zpad = jnp.zeros((_U, _DP - _D), jnp.float32)
    cat_u = jnp.concatenate([user_table, zpad], axis=1)
    cat_i = jnp.concatenate([item_table, zpad], axis=1)
    gu_idx, su_row, su_val = _prep_indices_half(uh_cols, uh_rows, uh_vals,
                                                _UCH, _U)
    gi_idx, si_row, si_val = _prep_indices_half(ih_cols, ih_rows, ih_vals,
                                                _ICH, _I)

    emb_u, emb_i = cat_u, cat_i
    msgs = []
    norm1_u = norm1_i = None
    final_u = final_i = None
    for l in range(_L):
        parts_u = _sc_segsum_half(emb_u, gu_idx, su_row, su_val, _UCH)
        parts_i = _sc_segsum_half(emb_i, gi_idx, si_row, si_val, _ICH)
        msg = _msg_mm(parts_u, parts_i, agg_W[l], agg_b[l])
        msgs.append(msg)
        if l == 0:
            norm1_u = _bigmm_pad_half(full_hyper, msg, 0)
            norm1_i = _bigmm_pad_half(full_hyper, msg, 1)
            emb_u, emb_i = norm1_u, norm1_i
        else:
            final_u = _bigmm_add2_half(full_hyper, msg, user_table,
                                       norm1_u, half=0)
            final_i = _bigmm_add2_half(full_hyper, msg, item_table,
                                       norm1_i, half=1)

    final_gp = _finalg_pad(overlap_graph, group_table, msgs[0], msgs[1])

    mem_idx = members.astype(jnp.int32).reshape(_NW, _NMCH, _BW)
    pos_idx = pos_item_inputs.astype(jnp.int32).reshape(_NW, _BW)
    neg_idx = neg_item_inputs.astype(jnp.int32).reshape(_NW, _BW)
    grp_idx = group_inputs.astype(jnp.int32).reshape(_NW, _BW)
    centers = _sc_memcen(final_u, mem_idx)
    i_pos, i_neg, g_b = _sc_pngb(final_i, final_gp, pos_idx, neg_idx,
                                 grp_idx)

    pred, parts = _batch_stage(centers, g_b, i_pos, i_neg, pW1, pb1, pW2, pb2)
    cl_loss = jnp.sum(parts[:, 0, 0]) / _B
    bpr_loss = jnp.sum(parts[:, 0, 1]) / _B
    loss = bpr_loss + cl_loss * _CL_W
    return (loss, pred)


# speedup vs baseline: 1.0433x; 1.0052x over previous
import jax
import jax.numpy as jnp
from jax import lax
from jax.experimental import pallas as pl
from jax.experimental.pallas import tpu as pltpu
from jax.experimental.pallas import tpu_sc as plsc

_U = 20000
_I = 20000
_G = 1000
_D = 64
_L = 2
_B = 4096
_M = 32
_TEMP = 0.2
_CL_W = 0.1

_NNZ = 96000
_NW = 32
_K = 128
_NCHUNK = 24
_NNZ_PAD = _NW * _NCHUNK * _K
_SEG = 2 * _G
_DP = 128

_ROWS_PER_BLK = 2000
_BATCH_TILE = 2048


_ZROWS = 200
_UCH = 9
_ICH = 18


def _seg_ring(emb_hbm, gidx_v, srow_v, vals_v, acc_sh, bufs, sems, ssems,
              nchunk):
    for b in range(2):
        pltpu.async_copy(emb_hbm.at[gidx_v.at[b]], bufs[b], sems[b])

    @pl.loop(0, nchunk, step=3)
    def _trip(c0):
        for b in range(3):
            c = c0 + b
            buf = bufs[b]
            p = (b + 2) % 3
            pltpu.make_async_copy(emb_hbm.at[gidx_v.at[c]], buf,
                                  sems[b]).wait()

            @pl.loop(0, _K // 16)
            def _scale(g):
                val16 = vals_v[c, pl.ds(g * 16, 16)]
                for l in range(16):
                    j = g * 16 + l
                    val = val16[l]
                    for c4 in range(_D // 16):
                        sl = pl.ds(c4 * 16, 16)
                        buf[j, sl] = buf[j, sl] * val

            pltpu.async_copy(buf, acc_sh.at[srow_v.at[c]], ssems[b],
                             add=True)

            @pl.when(c + 2 < nchunk)
            def _prefetch():
                @pl.when(c >= 1)
                def _drain_prev():
                    pltpu.make_async_copy(
                        bufs[p], acc_sh.at[srow_v.at[c]],
                        ssems[p]).wait()

                pltpu.async_copy(emb_hbm.at[gidx_v.at[c + 2]], bufs[p],
                                 sems[p])

    for b in range(3):
        pltpu.make_async_copy(bufs[b],
                              acc_sh.at[srow_v.at[nchunk - 3 + b]],
                              ssems[b]).wait()


def _make_segsum_body(nchunk, nseg):
    def _segsum_body(emb_hbm, gidx_hbm, srow_hbm, vals_hbm, out_hbm,
                     gidx_v, srow_v, vals_v, buf0, buf1, buf2, zbuf, acc_sh,
                     sem0, sem1, sem2, ssem0, ssem1, ssem2):
        cid = lax.axis_index("c")
        sid = lax.axis_index("s")
        wid = sid * 2 + cid
        bufs = (buf0, buf1, buf2)
        sems = (sem0, sem1, sem2)
        ssems = (ssem0, ssem1, ssem2)

        pltpu.sync_copy(gidx_hbm.at[wid], gidx_v)
        pltpu.sync_copy(srow_hbm.at[wid], srow_v)
        pltpu.sync_copy(vals_hbm.at[wid], vals_v)

        zeros16 = jnp.zeros((16,), jnp.float32)
        zw = nseg // _ZROWS

        @pl.when(sid < zw)
        def _zero_acc():
            @pl.loop(0, _ZROWS)
            def _zero(r):
                for c8 in range(_DP // 16):
                    zbuf[r, pl.ds(c8 * 16, 16)] = zeros16

            pltpu.sync_copy(zbuf, acc_sh.at[pl.ds(sid * _ZROWS, _ZROWS)])

        plsc.subcore_barrier()

        _seg_ring(emb_hbm, gidx_v, srow_v, vals_v, acc_sh, bufs, sems,
                  ssems, nchunk)

        plsc.subcore_barrier()

        @pl.when(sid == 0)
        def _flush():
            pltpu.sync_copy(acc_sh, out_hbm.at[cid])

    return _segsum_body


def _sc_segsum_half(emb, gidx, srow, vals, nchunk):
    return pl.kernel(
        _make_segsum_body(nchunk, _G),
        out_type=jax.ShapeDtypeStruct((2, _G, _DP), jnp.float32),
        mesh=plsc.VectorSubcoreMesh(core_axis_name="c", subcore_axis_name="s",
                                    num_cores=2, num_subcores=16),
        scratch_types=[
            pltpu.VMEM((nchunk, _K), jnp.int32),
            pltpu.VMEM((nchunk, _K), jnp.int32),
            pltpu.VMEM((nchunk, _K), jnp.float32),
            pltpu.VMEM((_K, _DP), jnp.float32),
            pltpu.VMEM((_K, _DP), jnp.float32),
            pltpu.VMEM((_K, _DP), jnp.float32),
            pltpu.VMEM((_ZROWS, _DP), jnp.float32),
            pltpu.VMEM_SHARED((_G, _DP), jnp.float32),
            pltpu.SemaphoreType.DMA,
            pltpu.SemaphoreType.DMA,
            pltpu.SemaphoreType.DMA,
            pltpu.SemaphoreType.DMA,
            pltpu.SemaphoreType.DMA,
            pltpu.SemaphoreType.DMA,
        ],
    )(emb, gidx, srow, vals)


def _prep_indices_half(cols, rows, vals, nchunk, nrows):
    n = cols.shape[0]
    pad = _NW * nchunk * _K - n
    pad_idx = jnp.arange(pad, dtype=jnp.int32) % nrows
    pad_row = jnp.arange(pad, dtype=jnp.int32) % _G
    gidx = jnp.concatenate([cols.astype(jnp.int32), pad_idx]
                           ).reshape(_NW, nchunk, _K)
    srow = jnp.concatenate([rows.astype(jnp.int32), pad_row]
                           ).reshape(_NW, nchunk, _K)
    svals = jnp.concatenate([vals, jnp.zeros((pad,), jnp.float32)]
                            ).reshape(_NW, nchunk, _K)
    return gidx, srow, svals


_BW = _B // _NW
_MCH = 4
_NMCH = _BW // _MCH


def _memcen_body(fui_hbm, mem_hbm, cen_out,
                 mem_idx, buf0, buf1, cbuf, sem0, sem1):
    cid = lax.axis_index("c")
    sid = lax.axis_index("s")
    wid = sid * 2 + cid
    bufs = (buf0, buf1)
    sems = (sem0, sem1)

    pltpu.sync_copy(mem_hbm.at[wid], mem_idx)

    for b in range(2):
        pltpu.async_copy(fui_hbm.at[mem_idx.at[b]], bufs[b], sems[b])

    @pl.loop(0, _NMCH, step=2)
    def _mpair(c0):
        for b in range(2):
            c = c0 + b
            buf = bufs[b]
            pltpu.make_async_copy(fui_hbm.at[mem_idx.at[c]], buf,
                                  sems[b]).wait()
            for g4 in range(_MCH):
                base = g4 * _M
                for c4 in range(_D // 16):
                    sl = pl.ds(c4 * 16, 16)
                    amax = buf[base, sl]
                    amin = amax
                    for r in range(1, _M):
                        x = buf[base + r, sl]
                        amax = jnp.maximum(amax, x)
                        amin = jnp.minimum(amin, x)
                    cbuf[c * _MCH + g4, sl] = (amax + amin) * 0.5

            @pl.when(c < _NMCH - 2)
            def _prefetch():
                pltpu.async_copy(fui_hbm.at[mem_idx.at[c + 2]], buf, sems[b])

    pltpu.sync_copy(cbuf, cen_out.at[pl.ds(wid * _BW, _BW)])


def _sc_memcen(fui_u, members):
    return pl.kernel(
        _memcen_body,
        out_type=jax.ShapeDtypeStruct((_B, _DP), jnp.float32),
        cost_estimate=pl.CostEstimate(
            flops=2 * _B * _M * _D, transcendentals=0,
            bytes_accessed=_B * _M * _DP * 4 + _B * _DP * 4),
        mesh=plsc.VectorSubcoreMesh(core_axis_name="c", subcore_axis_name="s",
                                    num_cores=2, num_subcores=16),
        scratch_types=[
            pltpu.VMEM((_NMCH, _BW), jnp.int32),
            pltpu.VMEM((_BW, _DP), jnp.float32),
            pltpu.VMEM((_BW, _DP), jnp.float32),
            pltpu.VMEM((_BW, _DP), jnp.float32),
            pltpu.SemaphoreType.DMA,
            pltpu.SemaphoreType.DMA,
        ],
    )(fui_u, members)


def _pngb_body(fui_hbm, fg_hbm, pos_hbm, neg_hbm, grp_hbm,
               pos_out, neg_out, gb_out,
               png_idx, buf0, buf1, sem0, sem1):
    cid = lax.axis_index("c")
    sid = lax.axis_index("s")
    wid = sid * 2 + cid

    pltpu.sync_copy(pos_hbm.at[wid], png_idx.at[0])
    pltpu.sync_copy(neg_hbm.at[wid], png_idx.at[1])
    pltpu.sync_copy(grp_hbm.at[wid], png_idx.at[2])

    pltpu.async_copy(fui_hbm.at[png_idx.at[0]], buf0, sem0)
    pltpu.async_copy(fui_hbm.at[png_idx.at[1]], buf1, sem1)
    pltpu.make_async_copy(fui_hbm.at[png_idx.at[0]], buf0, sem0).wait()
    pltpu.sync_copy(buf0, pos_out.at[pl.ds(wid * _BW, _BW)])
    pltpu.async_copy(fg_hbm.at[png_idx.at[2]], buf0, sem0)
    pltpu.make_async_copy(fui_hbm.at[png_idx.at[1]], buf1, sem1).wait()
    pltpu.sync_copy(buf1, neg_out.at[pl.ds(wid * _BW, _BW)])
    pltpu.make_async_copy(fg_hbm.at[png_idx.at[2]], buf0, sem0).wait()
    pltpu.sync_copy(buf0, gb_out.at[pl.ds(wid * _BW, _BW)])


def _sc_pngb(fui_i, fgp, pos_idx, neg_idx, grp_idx):
    out = jax.ShapeDtypeStruct((_B, _DP), jnp.float32)
    return pl.kernel(
        _pngb_body,
        out_type=[out, out, out],
        mesh=plsc.VectorSubcoreMesh(core_axis_name="c", subcore_axis_name="s",
                                    num_cores=2, num_subcores=16),
        scratch_types=[
            pltpu.VMEM((3, _BW), jnp.int32),
            pltpu.VMEM((_BW, _DP), jnp.float32),
            pltpu.VMEM((_BW, _DP), jnp.float32),
            pltpu.SemaphoreType.DMA,
            pltpu.SemaphoreType.DMA,
        ],
    )(fui_i, fgp, pos_idx, neg_idx, grp_idx)


def _finalg_kernel(a_ref, x_ref, m1_ref, m2_ref, o_ref):
    ge = jnp.dot(a_ref[...], x_ref[...], preferred_element_type=jnp.float32)
    s = ge + m1_ref[...] + m2_ref[...]
    o_ref[...] = jnp.concatenate(
        [s, jnp.zeros((_G, _DP - _D), jnp.float32)], axis=1)


def _finalg_pad(overlap_graph, group_table, msg1, msg2):
    return pl.pallas_call(
        _finalg_kernel,
        out_shape=jax.ShapeDtypeStruct((_G, _DP), jnp.float32),
    )(overlap_graph, group_table, msg1, msg2)


def _msg_kernel(pu_ref, pi_ref, w_ref, b_ref, o_ref):
    um = pu_ref[0, :, :_D] + pu_ref[1, :, :_D]
    im = pi_ref[0, :, :_D] + pi_ref[1, :, :_D]
    acc = jnp.dot(um, w_ref[:_D, :], preferred_element_type=jnp.float32)
    acc += jnp.dot(im, w_ref[_D:, :], preferred_element_type=jnp.float32)
    o_ref[...] = acc + b_ref[...]


def _msg_mm(parts_u, parts_i, w, b):
    return pl.pallas_call(
        _msg_kernel,
        out_shape=jax.ShapeDtypeStruct((_G, _D), jnp.float32),
    )(parts_u, parts_i, w, b.reshape(1, _D))


def _bigmm_pad_kernel(a_ref, x_ref, o_ref):
    mm = jnp.dot(a_ref[...], x_ref[...], preferred_element_type=jnp.float32)
    o_ref[...] = jnp.concatenate(
        [mm, jnp.zeros((mm.shape[0], _DP - _D), jnp.float32)], axis=1)


def _bigmm_pad_half(a, x, half):
    m_half = _U
    k = a.shape[1]
    off = half * (m_half // _ROWS_PER_BLK)
    return pl.pallas_call(
        _bigmm_pad_kernel,
        grid=(m_half // _ROWS_PER_BLK,),
        in_specs=[
            pl.BlockSpec((_ROWS_PER_BLK, k), lambda i: (i + off, 0)),
            pl.BlockSpec((k, _D), lambda i: (0, 0)),
        ],
        out_specs=pl.BlockSpec((_ROWS_PER_BLK, _DP), lambda i: (i, 0)),
        out_shape=jax.ShapeDtypeStruct((m_half, _DP), jnp.float32),
    )(a, x)


def _bigmm_add2_kernel(a_ref, x_ref, b1_ref, b2_ref, o_ref):
    s = (b1_ref[...] + b2_ref[..., :_D] +
         jnp.dot(a_ref[...], x_ref[...], preferred_element_type=jnp.float32))
    o_ref[...] = jnp.concatenate(
        [s, jnp.zeros((s.shape[0], _DP - _D), jnp.float32)], axis=1)


def _bigmm_add2_half(a, x, base1_half, base2_half, half):
    m_half = _U
    k = a.shape[1]
    off = half * (m_half // _ROWS_PER_BLK)
    return pl.pallas_call(
        _bigmm_add2_kernel,
        grid=(m_half // _ROWS_PER_BLK,),
        in_specs=[
            pl.BlockSpec((_ROWS_PER_BLK, k), lambda i: (i + off, 0)),
            pl.BlockSpec((k, _D), lambda i: (0, 0)),
            pl.BlockSpec((_ROWS_PER_BLK, _D), lambda i: (i, 0)),
            pl.BlockSpec((_ROWS_PER_BLK, _DP), lambda i: (i, 0)),
        ],
        out_specs=pl.BlockSpec((_ROWS_PER_BLK, _DP), lambda i: (i, 0)),
        out_shape=jax.ShapeDtypeStruct((m_half, _DP), jnp.float32),
    )(a, x, base1_half, base2_half)


def _batch_kernel(centers_ref, gb_ref, ipos_ref, ineg_ref,
                  pw1_ref, pb1_ref, pw2_ref, pb2_ref,
                  pred_ref, part_ref):
    i0 = pl.program_id(0)
    c = centers_ref[..., :_D]
    gb_all = gb_ref[..., :_D]
    gbt = gb_ref[pl.ds(i0 * _BATCH_TILE, _BATCH_TILE), :_D]

    v1 = c / (jnp.sqrt(jnp.sum(c * c, axis=1, keepdims=True)) + 1e-12)
    v2 = gb_all / (jnp.sqrt(jnp.sum(gb_all * gb_all, axis=1,
                                    keepdims=True)) + 1e-12)
    v2t = gbt / (jnp.sqrt(jnp.sum(gbt * gbt, axis=1, keepdims=True)) + 1e-12)

    scores = jnp.exp(jnp.dot(v1, v2.T, preferred_element_type=jnp.float32)
                     / _TEMP)
    ttl = jnp.sum(scores, axis=1)
    pos = jnp.exp(jnp.sum(v1 * v2t, axis=1) / _TEMP)
    cl_part = jnp.sum(jnp.log(ttl) - jnp.log(pos))

    def predict(x):
        h = jnp.dot(x, pw1_ref[...], preferred_element_type=jnp.float32)
        h = h + pb1_ref[...]
        h = jnp.where(h > 0, h, 0.01 * h)
        return jnp.dot(h, pw2_ref[...],
                       preferred_element_type=jnp.float32) + pb2_ref[...]

    spos = jax.nn.sigmoid(predict(gbt * ipos_ref[..., :_D]))
    sneg = jax.nn.sigmoid(predict(gbt * ineg_ref[..., :_D]))
    bpr_part = jnp.sum(jnp.log(1.0 + jnp.exp(sneg - spos)))

    pred_ref[...] = spos
    lane = jax.lax.broadcasted_iota(jnp.int32, (1, 128), 1)
    vec = jnp.where(lane == 0, cl_part,
                    jnp.where(lane == 1, bpr_part, 0.0))
    part_ref[...] = vec.reshape(1, 1, 128)


def _batch_stage(centers, g_b, i_pos, i_neg, pW1, pb1, pW2, pb2):
    nblk = _B // _BATCH_TILE
    tile = pl.BlockSpec((_BATCH_TILE, _DP), lambda i: (i, 0))
    full = pl.BlockSpec((_B, _DP), lambda i: (0, 0))
    pred, parts = pl.pallas_call(
        _batch_kernel,
        grid=(nblk,),
        in_specs=[
            tile, full, tile, tile,
            pl.BlockSpec((_D, 8), lambda i: (0, 0)),
            pl.BlockSpec((1, 8), lambda i: (0, 0)),
            pl.BlockSpec((8, 1), lambda i: (0, 0)),
            pl.BlockSpec((1, 1), lambda i: (0, 0)),
        ],
        out_specs=[
            pl.BlockSpec((_BATCH_TILE, 1), lambda i: (i, 0)),
            pl.BlockSpec((1, 1, 128), lambda i: (i, 0, 0)),
        ],
        out_shape=[
            jax.ShapeDtypeStruct((_B, 1), jnp.float32),
            jax.ShapeDtypeStruct((nblk, 1, 128), jnp.float32),
        ],
    )(centers, g_b, i_pos, i_neg, pW1, pb1.reshape(1, 8),
      pW2, pb2.reshape(1, 1))
    return pred, parts


def kernel(user_table, item_table, group_table, overlap_graph, full_hyper,
           uh_vals, ih_vals, agg_W, agg_b, pW1, pb1, pW2, pb2,
           group_inputs, pos_item_inputs, neg_item_inputs, members,
           uh_rows, uh_cols, ih_rows, ih_cols):
    zpad = jnp.zeros((_U, _DP - _D), jnp.float32)
    cat_u = jnp.concatenate([user_table, zpad], axis=1)
    cat_i = jnp.concatenate([item_table, zpad], axis=1)
    gu_idx, su_row, su_val = _prep_indices_half(uh_cols, uh_rows, uh_vals,
                                                _UCH, _U)
    gi_idx, si_row, si_val = _prep_indices_half(ih_cols, ih_rows, ih_vals,
                                                _ICH, _I)

    emb_u, emb_i = cat_u, cat_i
    msgs = []
    norm1_u = norm1_i = None
    final_u = final_i = None
    for l in range(_L):
        parts_u = _sc_segsum_half(emb_u, gu_idx, su_row, su_val, _UCH)
        parts_i = _sc_segsum_half(emb_i, gi_idx, si_row, si_val, _ICH)
        msg = _msg_mm(parts_u, parts_i, agg_W[l], agg_b[l])
        msgs.append(msg)
        if l == 0:
            norm1_u = _bigmm_pad_half(full_hyper, msg, 0)
            norm1_i = _bigmm_pad_half(full_hyper, msg, 1)
            emb_u, emb_i = norm1_u, norm1_i
        else:
            final_u = _bigmm_add2_half(full_hyper, msg, user_table,
                                       norm1_u, half=0)
            final_i = _bigmm_add2_half(full_hyper, msg, item_table,
                                       norm1_i, half=1)

    final_gp = _finalg_pad(overlap_graph, group_table, msgs[0], msgs[1])

    mem_idx = members.astype(jnp.int32).reshape(_NW, _NMCH, _BW)
    pos_idx = pos_item_inputs.astype(jnp.int32).reshape(_NW, _BW)
    neg_idx = neg_item_inputs.astype(jnp.int32).reshape(_NW, _BW)
    grp_idx = group_inputs.astype(jnp.int32).reshape(_NW, _BW)
    centers = _sc_memcen(final_u, mem_idx)
    i_pos, i_neg, g_b = _sc_pngb(final_i, final_gp, pos_idx, neg_idx,
                                 grp_idx)

    pred, parts = _batch_stage(centers, g_b, i_pos, i_neg, pW1, pb1, pW2, pb2)
    cl_loss = jnp.sum(parts[:, 0, 0]) / _B
    bpr_loss = jnp.sum(parts[:, 0, 1]) / _B
    loss = bpr_loss + cl_loss * _CL_W
    return (loss, pred)
